# baseline (speedup 1.0000x reference)
import functools

import jax
import jax.numpy as jnp
from jax import lax
from jax.experimental import pallas as pl
from jax.experimental.pallas import tpu as pltpu
from jax.experimental.pallas import tpu_sc as plsc

_N = 10000
_E = 320000
_D = 128
_DH = 64
_CW = 8

_NC = 2
_NS = 16
_BATCH = 125
_BPW = _E // (_BATCH * _NS)
_GRP = 8
_NG = _BPW // _GRP
_RPT = _N // _NS

_mesh = plsc.VectorSubcoreMesh(
    core_axis_name="c", subcore_axis_name="s", num_cores=_NC, num_subcores=_NS
)


@functools.partial(
    pl.kernel,
    out_type=[
        jax.ShapeDtypeStruct((_N, _NC, _DH), jnp.float32),
        jax.ShapeDtypeStruct((_NC, _N, _CW), jnp.float32),
    ],
    mesh=_mesh,
    compiler_params=pltpu.CompilerParams(use_tc_tiling_on_sc=False,
                                         skip_device_barrier=True),
    scratch_types=[
        pltpu.VMEM_SHARED((_N, _DH), jnp.float32),
        pltpu.VMEM_SHARED((_N, _DH), jnp.float32),
        pltpu.VMEM_SHARED((_N, _CW), jnp.float32),
        pltpu.VMEM((2, _GRP, _BATCH), jnp.int32),
        pltpu.VMEM((2, _GRP, _BATCH), jnp.int32),
        pltpu.VMEM((2, _BATCH, _DH), jnp.float32),
        pltpu.VMEM((_BATCH, _CW), jnp.float32),
        pltpu.SemaphoreType.DMA,
        pltpu.SemaphoreType.DMA,
    ],
)
def _sc_aggregate(src3d, dst3d, xsplit, zfeat, zcnt, ones8, out_sum, out_cnt,
                  x_sh, acc_sh, cnt_sh, srcg_v, dstg_v, rows_v, ones_v,
                  sem0, sem1):
    c = lax.axis_index("c")
    s = lax.axis_index("s")
    sems = (sem0, sem1)

    r0 = s * _RPT
    pltpu.sync_copy(xsplit.at[pl.ds(r0, _RPT), c], x_sh.at[pl.ds(r0, _RPT)])
    pltpu.sync_copy(zcnt, cnt_sh.at[pl.ds(r0, _RPT)])
    pltpu.sync_copy(ones8, ones_v)
    pltpu.sync_copy(zfeat, rows_v.at[0])
    for j in range(_RPT // _BATCH):
        pltpu.sync_copy(rows_v.at[0],
                        acc_sh.at[pl.ds(r0 + j * _BATCH, _BATCH)])

    pltpu.sync_copy(src3d.at[s, pl.ds(0, _GRP)], srcg_v.at[0])
    pltpu.sync_copy(dst3d.at[s, pl.ds(0, _GRP)], dstg_v.at[0])
    plsc.subcore_barrier()
    for j in range(2):
        pltpu.async_copy(x_sh.at[srcg_v.at[0, j]], rows_v.at[j], sems[j])

    def wait_gather(slot):
        pltpu.make_async_copy(x_sh.at[srcg_v.at[0, 0]], rows_v.at[slot],
                              sems[slot]).wait()

    def group(g, last):
        gp = lax.rem(g, 2)
        if not last:
            pltpu.sync_copy(src3d.at[s, pl.ds((g + 1) * _GRP, _GRP)],
                            srcg_v.at[1 - gp])
            pltpu.sync_copy(dst3d.at[s, pl.ds((g + 1) * _GRP, _GRP)],
                            dstg_v.at[1 - gp])
        for j in range(_GRP):
            slot = j % 2
            wait_gather(slot)
            pltpu.sync_copy(rows_v.at[slot], acc_sh.at[dstg_v.at[gp, j]],
                            add=True)
            pltpu.sync_copy(ones_v, cnt_sh.at[dstg_v.at[gp, j]], add=True)
            if j < _GRP - 2:
                pltpu.async_copy(x_sh.at[srcg_v.at[gp, j + 2]],
                                 rows_v.at[slot], sems[slot])
            elif not last:
                pltpu.async_copy(x_sh.at[srcg_v.at[1 - gp, j - (_GRP - 2)]],
                                 rows_v.at[slot], sems[slot])

    lax.fori_loop(0, _NG - 1, lambda g, _: (group(g, False), 0)[1], 0)
    group(_NG - 1, True)
    plsc.subcore_barrier()

    pltpu.sync_copy(acc_sh.at[pl.ds(r0, _RPT)],
                    out_sum.at[pl.ds(r0, _RPT), c])
    pltpu.sync_copy(cnt_sh.at[pl.ds(r0, _RPT)],
                    out_cnt.at[c, pl.ds(r0, _RPT)])


_R = 1000


def _tc_heads_body(sum_ref, cnt_ref, w1m, b1m, w1v, b1v, wmo, bmo, wvo, bvo,
                   mean_ref, var_ref):
    cnt = cnt_ref[0, :, :1]
    agg = sum_ref[...] / jnp.maximum(cnt, 1.0)
    hm = jnp.maximum(
        jnp.dot(agg, w1m[...], preferred_element_type=jnp.float32) + b1m[...],
        0.0)
    mean_ref[...] = (
        jnp.dot(hm, wmo[...], preferred_element_type=jnp.float32) + bmo[...])
    hv = jnp.maximum(
        jnp.dot(agg, w1v[...], preferred_element_type=jnp.float32) + b1v[...],
        0.0)
    var_ref[...] = (
        jnp.dot(hv, wvo[...], preferred_element_type=jnp.float32) + bvo[...])


def _tc_heads(sums, cnt, W1m, b1m, W1v, b1v, Wmo, bmo, Wvo, bvo):
    wspec = pl.BlockSpec((_D, _D), lambda i: (0, 0))
    bspec = pl.BlockSpec((1, _D), lambda i: (0, 0))
    return pl.pallas_call(
        _tc_heads_body,
        grid=(_N // _R,),
        in_specs=[
            pl.BlockSpec((_R, _D), lambda i: (i, 0)),
            pl.BlockSpec((1, _R, _CW), lambda i: (0, i, 0)),
            wspec, bspec, wspec, bspec, wspec, bspec, wspec, bspec,
        ],
        out_specs=[
            pl.BlockSpec((_R, _D), lambda i: (i, 0)),
            pl.BlockSpec((_R, _D), lambda i: (i, 0)),
        ],
        out_shape=[
            jax.ShapeDtypeStruct((_N, _D), jnp.float32),
            jax.ShapeDtypeStruct((_N, _D), jnp.float32),
        ],
    )(sums, cnt, W1m, b1m, W1v, b1v, Wmo, bmo, Wvo, bvo)


@jax.jit
def kernel(x, edge_index, W1_mean, b1_mean, W1_var, b1_var,
           W_mean_out, b_mean_out, W_var_out, b_var_out):
    src3d = edge_index[0].reshape(_NS, _BPW, _BATCH)
    dst3d = edge_index[1].reshape(_NS, _BPW, _BATCH)
    xsplit = x.reshape(_N, _NC, _DH)
    zfeat = jnp.zeros((_BATCH, _DH), jnp.float32)
    zcnt = jnp.zeros((_RPT, _CW), jnp.float32)
    ones8 = jnp.ones((_BATCH, _CW), jnp.float32)
    sums, cnt = _sc_aggregate(src3d, dst3d, xsplit, zfeat, zcnt, ones8)
    mean, variance = _tc_heads(
        sums.reshape(_N, _D), cnt, W1_mean, b1_mean.reshape(1, _D), W1_var,
        b1_var.reshape(1, _D), W_mean_out, b_mean_out.reshape(1, _D),
        W_var_out, b_var_out.reshape(1, _D))
    return (mean, variance)


# speedup vs baseline: 1.7945x; 1.7945x over previous
import functools

import jax
import jax.numpy as jnp
from jax import lax
from jax.experimental import pallas as pl
from jax.experimental.pallas import tpu as pltpu
from jax.experimental.pallas import tpu_sc as plsc

_N = 10000
_E = 320000
_D = 128
_CW = 8

_NC = 2
_NS = 16
_NW = _NC * _NS
_BATCH = 125
_BPW = (_E // _BATCH) // _NW
_GRP = 4
_NG = _BPW // _GRP
_RPT = _N // _NS

_mesh = plsc.VectorSubcoreMesh(
    core_axis_name="c", subcore_axis_name="s", num_cores=_NC, num_subcores=_NS
)


@functools.partial(
    pl.kernel,
    out_type=[
        jax.ShapeDtypeStruct((_NC, _N, _D), jnp.float32),
        jax.ShapeDtypeStruct((_NC, _N, _CW), jnp.float32),
    ],
    mesh=_mesh,
    compiler_params=pltpu.CompilerParams(use_tc_tiling_on_sc=False,
                                         skip_device_barrier=True),
    scratch_types=[
        pltpu.VMEM_SHARED((_N, _D), jnp.float32),
        pltpu.VMEM_SHARED((_N, _CW), jnp.float32),
        pltpu.VMEM((2, _GRP, _BATCH), jnp.int32),
        pltpu.VMEM((2, _GRP, _BATCH), jnp.int32),
        pltpu.VMEM((2, _BATCH, _D), jnp.float32),
        pltpu.VMEM((_BATCH, _CW), jnp.float32),
        pltpu.SemaphoreType.DMA,
        pltpu.SemaphoreType.DMA,
    ],
)
def _sc_aggregate(src3d, dst3d, x, zfeat, zcnt, ones8, out_sum, out_cnt,
                  acc_sh, cnt_sh, srcg_v, dstg_v, rows_v, ones_v, sem0, sem1):
    c = lax.axis_index("c")
    s = lax.axis_index("s")
    wid = s * _NC + c
    sems = (sem0, sem1)

    r0 = s * _RPT
    pltpu.sync_copy(zcnt, cnt_sh.at[pl.ds(r0, _RPT)])
    pltpu.sync_copy(ones8, ones_v)
    pltpu.sync_copy(zfeat, rows_v.at[0])
    for j in range(_RPT // _BATCH):
        pltpu.sync_copy(rows_v.at[0],
                        acc_sh.at[pl.ds(r0 + j * _BATCH, _BATCH)])

    pltpu.sync_copy(src3d.at[wid, pl.ds(0, _GRP)], srcg_v.at[0])
    pltpu.sync_copy(dst3d.at[wid, pl.ds(0, _GRP)], dstg_v.at[0])
    for j in range(2):
        pltpu.async_copy(x.at[srcg_v.at[0, j]], rows_v.at[j], sems[j])
    plsc.subcore_barrier()

    def wait_gather(slot):
        pltpu.make_async_copy(x.at[srcg_v.at[0, 0]], rows_v.at[slot],
                              sems[slot]).wait()

    def group(g, last):
        gp = lax.rem(g, 2)
        if not last:
            pltpu.sync_copy(src3d.at[wid, pl.ds((g + 1) * _GRP, _GRP)],
                            srcg_v.at[1 - gp])
            pltpu.sync_copy(dst3d.at[wid, pl.ds((g + 1) * _GRP, _GRP)],
                            dstg_v.at[1 - gp])
        for j in range(_GRP):
            slot = j % 2
            wait_gather(slot)
            pltpu.sync_copy(rows_v.at[slot], acc_sh.at[dstg_v.at[gp, j]],
                            add=True)
            pltpu.sync_copy(ones_v, cnt_sh.at[dstg_v.at[gp, j]], add=True)
            if j < _GRP - 2:
                pltpu.async_copy(x.at[srcg_v.at[gp, j + 2]],
                                 rows_v.at[slot], sems[slot])
            elif not last:
                pltpu.async_copy(x.at[srcg_v.at[1 - gp, j - (_GRP - 2)]],
                                 rows_v.at[slot], sems[slot])

    lax.fori_loop(0, _NG - 1, lambda g, _: (group(g, False), 0)[1], 0)
    group(_NG - 1, True)
    plsc.subcore_barrier()

    pltpu.sync_copy(acc_sh.at[pl.ds(r0, _RPT)],
                    out_sum.at[c, pl.ds(r0, _RPT)])
    pltpu.sync_copy(cnt_sh.at[pl.ds(r0, _RPT)],
                    out_cnt.at[c, pl.ds(r0, _RPT)])


_R = 1000


def _tc_heads_body(sum_ref, cnt_ref, w1m, b1m, w1v, b1v, wmo, bmo, wvo, bvo,
                   mean_ref, var_ref):
    sums = sum_ref[0] + sum_ref[1]
    cnt = cnt_ref[0, :, :1] + cnt_ref[1, :, :1]
    agg = sums / jnp.maximum(cnt, 1.0)
    hm = jnp.maximum(
        jnp.dot(agg, w1m[...], preferred_element_type=jnp.float32) + b1m[...],
        0.0)
    mean_ref[...] = (
        jnp.dot(hm, wmo[...], preferred_element_type=jnp.float32) + bmo[...])
    hv = jnp.maximum(
        jnp.dot(agg, w1v[...], preferred_element_type=jnp.float32) + b1v[...],
        0.0)
    var_ref[...] = (
        jnp.dot(hv, wvo[...], preferred_element_type=jnp.float32) + bvo[...])


def _tc_heads(sums, cnt, W1m, b1m, W1v, b1v, Wmo, bmo, Wvo, bvo):
    wspec = pl.BlockSpec((_D, _D), lambda i: (0, 0))
    bspec = pl.BlockSpec((1, _D), lambda i: (0, 0))
    return pl.pallas_call(
        _tc_heads_body,
        grid=(_N // _R,),
        in_specs=[
            pl.BlockSpec((_NC, _R, _D), lambda i: (0, i, 0)),
            pl.BlockSpec((_NC, _R, _CW), lambda i: (0, i, 0)),
            wspec, bspec, wspec, bspec, wspec, bspec, wspec, bspec,
        ],
        out_specs=[
            pl.BlockSpec((_R, _D), lambda i: (i, 0)),
            pl.BlockSpec((_R, _D), lambda i: (i, 0)),
        ],
        out_shape=[
            jax.ShapeDtypeStruct((_N, _D), jnp.float32),
            jax.ShapeDtypeStruct((_N, _D), jnp.float32),
        ],
    )(sums, cnt, W1m, b1m, W1v, b1v, Wmo, bmo, Wvo, bvo)


@jax.jit
def kernel(x, edge_index, W1_mean, b1_mean, W1_var, b1_var,
           W_mean_out, b_mean_out, W_var_out, b_var_out):
    src3d = edge_index[0].reshape(_NW, _BPW, _BATCH)
    dst3d = edge_index[1].reshape(_NW, _BPW, _BATCH)
    zfeat = jnp.zeros((_BATCH, _D), jnp.float32)
    zcnt = jnp.zeros((_RPT, _CW), jnp.float32)
    ones8 = jnp.ones((_BATCH, _CW), jnp.float32)
    sums, cnt = _sc_aggregate(src3d, dst3d, x, zfeat, zcnt, ones8)
    mean, variance = _tc_heads(
        sums, cnt, W1_mean, b1_mean.reshape(1, _D), W1_var,
        b1_var.reshape(1, _D), W_mean_out, b_mean_out.reshape(1, _D),
        W_var_out, b_var_out.reshape(1, _D))
    return (mean, variance)


# speedup vs baseline: 1.8694x; 1.0417x over previous
import functools

import jax
import jax.numpy as jnp
from jax import lax
from jax.experimental import pallas as pl
from jax.experimental.pallas import tpu as pltpu
from jax.experimental.pallas import tpu_sc as plsc

_N = 10000
_E = 320000
_D = 128
_CW = 8

_NC = 2
_NS = 16
_NW = _NC * _NS
_BATCH = 125
_BPW = (_E // _BATCH) // _NW
_GRP = 16
_NG = _BPW // _GRP
_RPT = _N // _NS

_mesh = plsc.VectorSubcoreMesh(
    core_axis_name="c", subcore_axis_name="s", num_cores=_NC, num_subcores=_NS
)


@functools.partial(
    pl.kernel,
    out_type=[
        jax.ShapeDtypeStruct((_NC, _N, _D), jnp.float32),
        jax.ShapeDtypeStruct((_NC, _N, _CW), jnp.float32),
    ],
    mesh=_mesh,
    compiler_params=pltpu.CompilerParams(use_tc_tiling_on_sc=False,
                                         skip_device_barrier=True),
    scratch_types=[
        pltpu.VMEM_SHARED((_N, _D), jnp.float32),
        pltpu.VMEM_SHARED((_N, _CW), jnp.float32),
        pltpu.VMEM((2, _GRP, _BATCH), jnp.int32),
        pltpu.VMEM((2, _GRP, _BATCH), jnp.int32),
        pltpu.VMEM((2, _BATCH, _D), jnp.float32),
        pltpu.VMEM((_BATCH, _CW), jnp.float32),
        pltpu.SemaphoreType.DMA,
        pltpu.SemaphoreType.DMA,
    ],
)
def _sc_aggregate(src3d, dst3d, x, zfeat, zcnt, ones8, out_sum, out_cnt,
                  acc_sh, cnt_sh, srcg_v, dstg_v, rows_v, ones_v, sem0, sem1):
    c = lax.axis_index("c")
    s = lax.axis_index("s")
    wid = s * _NC + c
    sems = (sem0, sem1)

    r0 = s * _RPT
    pltpu.sync_copy(zcnt, cnt_sh.at[pl.ds(r0, _RPT)])
    pltpu.sync_copy(ones8, ones_v)
    pltpu.sync_copy(zfeat, rows_v.at[0])
    for j in range(_RPT // _BATCH):
        pltpu.sync_copy(rows_v.at[0],
                        acc_sh.at[pl.ds(r0 + j * _BATCH, _BATCH)])

    pltpu.sync_copy(src3d.at[wid, pl.ds(0, _GRP)], srcg_v.at[0])
    pltpu.sync_copy(dst3d.at[wid, pl.ds(0, _GRP)], dstg_v.at[0])
    for j in range(2):
        pltpu.async_copy(x.at[srcg_v.at[0, j]], rows_v.at[j], sems[j])
    plsc.subcore_barrier()

    def wait_gather(slot):
        pltpu.make_async_copy(x.at[srcg_v.at[0, 0]], rows_v.at[slot],
                              sems[slot]).wait()

    def group(g, last):
        gp = lax.rem(g, 2)
        if not last:
            pltpu.sync_copy(src3d.at[wid, pl.ds((g + 1) * _GRP, _GRP)],
                            srcg_v.at[1 - gp])
            pltpu.sync_copy(dst3d.at[wid, pl.ds((g + 1) * _GRP, _GRP)],
                            dstg_v.at[1 - gp])
        for j in range(_GRP):
            slot = j % 2
            wait_gather(slot)
            pltpu.sync_copy(rows_v.at[slot], acc_sh.at[dstg_v.at[gp, j]],
                            add=True)
            pltpu.sync_copy(ones_v, cnt_sh.at[dstg_v.at[gp, j]], add=True)
            if j < _GRP - 2:
                pltpu.async_copy(x.at[srcg_v.at[gp, j + 2]],
                                 rows_v.at[slot], sems[slot])
            elif not last:
                pltpu.async_copy(x.at[srcg_v.at[1 - gp, j - (_GRP - 2)]],
                                 rows_v.at[slot], sems[slot])

    lax.fori_loop(0, _NG - 1, lambda g, _: (group(g, False), 0)[1], 0)
    group(_NG - 1, True)
    plsc.subcore_barrier()

    pltpu.sync_copy(acc_sh.at[pl.ds(r0, _RPT)],
                    out_sum.at[c, pl.ds(r0, _RPT)])
    pltpu.sync_copy(cnt_sh.at[pl.ds(r0, _RPT)],
                    out_cnt.at[c, pl.ds(r0, _RPT)])


_R = 1000


def _tc_heads_body(sum_ref, cnt_ref, w1m, b1m, w1v, b1v, wmo, bmo, wvo, bvo,
                   mean_ref, var_ref):
    sums = sum_ref[0] + sum_ref[1]
    cnt = cnt_ref[0, :, :1] + cnt_ref[1, :, :1]
    agg = sums / jnp.maximum(cnt, 1.0)
    hm = jnp.maximum(
        jnp.dot(agg, w1m[...], preferred_element_type=jnp.float32) + b1m[...],
        0.0)
    mean_ref[...] = (
        jnp.dot(hm, wmo[...], preferred_element_type=jnp.float32) + bmo[...])
    hv = jnp.maximum(
        jnp.dot(agg, w1v[...], preferred_element_type=jnp.float32) + b1v[...],
        0.0)
    var_ref[...] = (
        jnp.dot(hv, wvo[...], preferred_element_type=jnp.float32) + bvo[...])


def _tc_heads(sums, cnt, W1m, b1m, W1v, b1v, Wmo, bmo, Wvo, bvo):
    wspec = pl.BlockSpec((_D, _D), lambda i: (0, 0))
    bspec = pl.BlockSpec((1, _D), lambda i: (0, 0))
    return pl.pallas_call(
        _tc_heads_body,
        grid=(_N // _R,),
        in_specs=[
            pl.BlockSpec((_NC, _R, _D), lambda i: (0, i, 0)),
            pl.BlockSpec((_NC, _R, _CW), lambda i: (0, i, 0)),
            wspec, bspec, wspec, bspec, wspec, bspec, wspec, bspec,
        ],
        out_specs=[
            pl.BlockSpec((_R, _D), lambda i: (i, 0)),
            pl.BlockSpec((_R, _D), lambda i: (i, 0)),
        ],
        out_shape=[
            jax.ShapeDtypeStruct((_N, _D), jnp.float32),
            jax.ShapeDtypeStruct((_N, _D), jnp.float32),
        ],
    )(sums, cnt, W1m, b1m, W1v, b1v, Wmo, bmo, Wvo, bvo)


@jax.jit
def kernel(x, edge_index, W1_mean, b1_mean, W1_var, b1_var,
           W_mean_out, b_mean_out, W_var_out, b_var_out):
    src3d = edge_index[0].reshape(_NW, _BPW, _BATCH)
    dst3d = edge_index[1].reshape(_NW, _BPW, _BATCH)
    zfeat = jnp.zeros((_BATCH, _D), jnp.float32)
    zcnt = jnp.zeros((_RPT, _CW), jnp.float32)
    ones8 = jnp.ones((_BATCH, _CW), jnp.float32)
    sums, cnt = _sc_aggregate(src3d, dst3d, x, zfeat, zcnt, ones8)
    mean, variance = _tc_heads(
        sums, cnt, W1_mean, b1_mean.reshape(1, _D), W1_var,
        b1_var.reshape(1, _D), W_mean_out, b_mean_out.reshape(1, _D),
        W_var_out, b_var_out.reshape(1, _D))
    return (mean, variance)
